# Initial kernel scaffold; baseline (speedup 1.0000x reference)
import jax
import jax.numpy as jnp
from jax.experimental import pallas as pl


def kernel(adjacency_matrices, weights_matrix, data, W):
    raise NotImplementedError("write your pallas kernel here")


# speedup vs baseline: 1.5357x; 1.5357x over previous
import jax
import jax.numpy as jnp
from jax.experimental import pallas as pl
from jax.experimental.pallas import tpu as pltpu

_K = 3
_BI = 256
_WPAD = 64


def _tagconv_body(a_ref, x0_ref, winit_ref, wmid_ref, out_ref, a_scr, xbuf):
    n = a_ref.shape[1]
    f = winit_ref.shape[1]
    k = pl.program_id(0)
    i = pl.program_id(1)
    row = pl.ds(i * _BI, _BI)

    @pl.when(jnp.logical_and(k == 0, i == 0))
    def _init():
        z = jnp.dot(x0_ref[...], winit_ref[...],
                    preferred_element_type=jnp.float32)
        ones = jnp.ones((n, 1), jnp.float32)
        zeros = jnp.zeros((n, _WPAD - f - 1), jnp.float32)
        xbuf[0] = jnp.concatenate([z, ones, zeros], axis=1).astype(jnp.bfloat16)

    @pl.when(k == 0)
    def _stage():
        a_scr[row, :] = a_ref[...].astype(jnp.bfloat16)

    def _hop(rd, wr, wsel):
        cur = xbuf[rd]
        a = a_scr[row, :]
        y = jnp.dot(a, cur, preferred_element_type=jnp.float32)
        y = y + xbuf[rd, row, :].astype(jnp.float32)
        rs = y[:, f:f + 1]
        rs = jnp.where(rs == 0.0, 1.0, rs)
        y = y / rs

        @pl.when(k < _K - 1)
        def _next():
            add = jnp.dot(x0_ref[row, :], wmid_ref[wsel],
                          preferred_element_type=jnp.float32)
            xbuf[wr, row, :] = (y + add).astype(jnp.bfloat16)

        out_ref[...] = y[:, :f]

    k2 = jax.lax.rem(k, 2)

    @pl.when(k2 == 0)
    def _even():
        _hop(0, 1, 0)

    @pl.when(k2 == 1)
    def _odd():
        _hop(1, 0, 1)


def kernel(adjacency_matrices, weights_matrix, data, W):
    del weights_matrix
    n = adjacency_matrices.shape[-1]
    c, f = W.shape[0], W.shape[1]
    nblk = n // _BI
    pad = jnp.zeros((c, _WPAD - f), jnp.float32)
    wmid = jnp.stack([
        jnp.concatenate([W[:, :, 2], pad], axis=1),
        jnp.concatenate([W[:, :, 0] + W[:, :, 1], pad], axis=1),
    ])
    winit = W[:, :, 3]

    return pl.pallas_call(
        _tagconv_body,
        grid=(_K, nblk),
        in_specs=[
            pl.BlockSpec((_BI, n),
                         lambda k, i: (jnp.where(k == 0, i, nblk - 1), 0)),
            pl.BlockSpec((n, c), lambda k, i: (0, 0)),
            pl.BlockSpec((c, f), lambda k, i: (0, 0)),
            pl.BlockSpec((2, c, _WPAD), lambda k, i: (0, 0, 0)),
        ],
        out_specs=pl.BlockSpec((_BI, f), lambda k, i: (i, 0)),
        out_shape=jax.ShapeDtypeStruct((n, f), jnp.float32),
        scratch_shapes=[
            pltpu.VMEM((n, n), jnp.bfloat16),
            pltpu.VMEM((2, n, _WPAD), jnp.bfloat16),
        ],
    )(adjacency_matrices, data, winit, wmid)


# speedup vs baseline: 1.8060x; 1.1760x over previous
import jax
import jax.numpy as jnp
from jax.experimental import pallas as pl
from jax.experimental.pallas import tpu as pltpu

_K = 3
_BI = 256
_WPAD = 64


def _tagconv_body(a_ref, x0t_ref, winit_ref, wmid_ref, out_ref, a_scr, xbuf):
    n = a_ref.shape[1]
    f = winit_ref.shape[0]
    k = pl.program_id(0)
    i = pl.program_id(1)
    row = pl.ds(i * _BI, _BI)

    @pl.when(jnp.logical_and(k == 0, i == 0))
    def _init():
        z = jnp.dot(winit_ref[...], x0t_ref[...],
                    preferred_element_type=jnp.float32)
        ones = jnp.ones((1, n), jnp.float32)
        zeros = jnp.zeros((_WPAD - f - 1, n), jnp.float32)
        xbuf[0] = jnp.concatenate([z, ones, zeros], axis=0).astype(jnp.bfloat16)

    @pl.when(k == 0)
    def _stage():
        a_scr[row, :] = a_ref[...].astype(jnp.bfloat16)

    def _hop(rd, wr, wsel):
        cur_t = xbuf[rd]
        a = a_scr[row, :]
        y = jax.lax.dot_general(cur_t, a, (((1,), (1,)), ((), ())),
                                preferred_element_type=jnp.float32)
        y = y + xbuf[rd, :, row].astype(jnp.float32)
        rs = y[f:f + 1, :]
        rs = jnp.where(rs == 0.0, 1.0, rs)
        y = y / rs

        @pl.when(k < _K - 1)
        def _next():
            add = jnp.dot(wmid_ref[wsel], x0t_ref[:, row],
                          preferred_element_type=jnp.float32)
            xbuf[wr, :, row] = (y + add).astype(jnp.bfloat16)

        out_ref[...] = y[:f, :]

    k2 = jax.lax.rem(k, 2)

    @pl.when(k2 == 0)
    def _even():
        _hop(0, 1, 0)

    @pl.when(k2 == 1)
    def _odd():
        _hop(1, 0, 1)


def kernel(adjacency_matrices, weights_matrix, data, W):
    del weights_matrix
    n = adjacency_matrices.shape[-1]
    c, f = W.shape[0], W.shape[1]
    nblk = n // _BI
    pad = jnp.zeros((c, _WPAD - f), jnp.float32)
    wmid_t = jnp.stack([
        jnp.concatenate([W[:, :, 2], pad], axis=1).T,
        jnp.concatenate([W[:, :, 0] + W[:, :, 1], pad], axis=1).T,
    ])
    winit_t = W[:, :, 3].T
    x0_t = data.T

    out_t = pl.pallas_call(
        _tagconv_body,
        grid=(_K, nblk),
        in_specs=[
            pl.BlockSpec((_BI, n),
                         lambda k, i: (jnp.where(k == 0, i, nblk - 1), 0)),
            pl.BlockSpec((c, n), lambda k, i: (0, 0)),
            pl.BlockSpec((f, c), lambda k, i: (0, 0)),
            pl.BlockSpec((2, _WPAD, c), lambda k, i: (0, 0, 0)),
        ],
        out_specs=pl.BlockSpec((f, _BI), lambda k, i: (0, i)),
        out_shape=jax.ShapeDtypeStruct((f, n), jnp.float32),
        scratch_shapes=[
            pltpu.VMEM((n, n), jnp.bfloat16),
            pltpu.VMEM((2, _WPAD, n), jnp.bfloat16),
        ],
    )(adjacency_matrices, x0_t, winit_t, wmid_t)
    return out_t.T


# speedup vs baseline: 2.0793x; 1.1513x over previous
import jax
import jax.numpy as jnp
from jax.experimental import pallas as pl
from jax.experimental.pallas import tpu as pltpu

_K = 3
_BI = 512
_WPAD = 64


def _tagconv_body(a_ref, x0t_ref, winit_ref, wmid_ref, out_ref, a_scr, xbuf):
    n = a_ref.shape[1]
    f = winit_ref.shape[0]
    k = pl.program_id(0)
    i = pl.program_id(1)
    row = pl.ds(i * _BI, _BI)

    @pl.when(jnp.logical_and(k == 0, i == 0))
    def _init():
        z = jnp.dot(winit_ref[...], x0t_ref[...],
                    preferred_element_type=jnp.float32)
        ones = jnp.ones((1, n), jnp.float32)
        zeros = jnp.zeros((_WPAD - f - 1, n), jnp.float32)
        xbuf[0] = jnp.concatenate([z, ones, zeros], axis=0).astype(jnp.bfloat16)

    @pl.when(k == 0)
    def _stage():
        a_scr[row, :] = a_ref[...].astype(jnp.bfloat16)

    def _hop(rd, wr, wsel):
        cur_t = xbuf[rd]
        a = a_scr[row, :]
        y = jax.lax.dot_general(cur_t, a, (((1,), (1,)), ((), ())),
                                preferred_element_type=jnp.float32)
        y = y + xbuf[rd, :, row].astype(jnp.float32)
        rs = y[f:f + 1, :]
        rs = jnp.where(rs == 0.0, 1.0, rs)
        y = y / rs

        @pl.when(k < _K - 1)
        def _next():
            add = jnp.dot(wmid_ref[wsel], x0t_ref[:, row],
                          preferred_element_type=jnp.float32)
            xbuf[wr, :, row] = (y + add).astype(jnp.bfloat16)

        out_ref[...] = y[:f, :]

    k2 = jax.lax.rem(k, 2)

    @pl.when(k2 == 0)
    def _even():
        _hop(0, 1, 0)

    @pl.when(k2 == 1)
    def _odd():
        _hop(1, 0, 1)


def kernel(adjacency_matrices, weights_matrix, data, W):
    del weights_matrix
    n = adjacency_matrices.shape[-1]
    c, f = W.shape[0], W.shape[1]
    nblk = n // _BI
    pad = jnp.zeros((c, _WPAD - f), jnp.float32)
    wmid_t = jnp.stack([
        jnp.concatenate([W[:, :, 2], pad], axis=1).T,
        jnp.concatenate([W[:, :, 0] + W[:, :, 1], pad], axis=1).T,
    ])
    winit_t = W[:, :, 3].T
    x0_t = data.T

    out_t = pl.pallas_call(
        _tagconv_body,
        grid=(_K, nblk),
        in_specs=[
            pl.BlockSpec((_BI, n),
                         lambda k, i: (jnp.where(k == 0, i, nblk - 1), 0)),
            pl.BlockSpec((c, n), lambda k, i: (0, 0)),
            pl.BlockSpec((f, c), lambda k, i: (0, 0)),
            pl.BlockSpec((2, _WPAD, c), lambda k, i: (0, 0, 0)),
        ],
        out_specs=pl.BlockSpec((f, _BI), lambda k, i: (0, i)),
        out_shape=jax.ShapeDtypeStruct((f, n), jnp.float32),
        scratch_shapes=[
            pltpu.VMEM((n, n), jnp.bfloat16),
            pltpu.VMEM((2, _WPAD, n), jnp.bfloat16),
        ],
    )(adjacency_matrices, x0_t, winit_t, wmid_t)
    return out_t.T
